# baseline (speedup 1.0000x reference)
import jax
import jax.numpy as jnp
from jax import lax
from jax.experimental import pallas as pl
from jax.experimental.pallas import tpu as pltpu

_MXU_ROWS = 256
_UNROLL = 32


def _batched_lstm_head_kernel(xt_ref, wih_ref, whh_ref, b_ref, wlin_ref,
                              blin_ref, out_ref):
    seq_len, batch, _ = xt_ref.shape
    Hp = whh_ref.shape[0]
    wdtype = whh_ref.dtype
    n_grp = max(1, batch // _MXU_ROWS)
    rows = batch // n_grp

    def step(t, carry):
        xt = xt_ref[t]
        new = []
        for j in range(n_grp):
            h, c = carry[2 * j], carry[2 * j + 1]
            pre = (jnp.dot(xt[j * rows:(j + 1) * rows], wih_ref[...],
                           preferred_element_type=jnp.float32)
                   + jnp.dot(h.astype(wdtype), whh_ref[...],
                             preferred_element_type=jnp.float32)
                   + b_ref[...])

            sig = jnp.tanh(pre[:, :3 * Hp] * 0.5) * 0.5 + 0.5
            i_g = sig[:, 0:Hp]
            f_g = sig[:, Hp:2 * Hp]
            o_g = sig[:, 2 * Hp:3 * Hp]
            g_g = jnp.tanh(pre[:, 3 * Hp:])

            c_new = f_g * c + i_g * g_g
            h_new = o_g * jnp.tanh(c_new)
            new += [h_new, c_new]
        return tuple(new)

    init = tuple(jnp.zeros((rows, Hp), jnp.float32) for _ in range(2 * n_grp))
    carry = lax.fori_loop(0, seq_len, step, init, unroll=_UNROLL)

    for j in range(n_grp):
        h_last = carry[2 * j]
        out_ref[j * rows:(j + 1) * rows, :] = (
            jnp.dot(h_last.astype(wlin_ref.dtype), wlin_ref[...],
                    preferred_element_type=jnp.float32) + blin_ref[...])


def _full_spec(arr):
    ndim = arr.ndim
    return pl.BlockSpec(arr.shape, lambda n: (0,) * ndim)


def _pick_batch(n_seq):
    for b in (256, 128, 64, 32, 16, 8):
        if n_seq % b == 0:
            return b
    return n_seq


@jax.jit
def kernel(xs, wih_f, whh_f, bias_f, wlin_f, blin_f):
    n_seq, seq_len, input_size = xs.shape
    output_size = wlin_f.shape[1]
    B = _pick_batch(n_seq)

    xt = jnp.transpose(xs.astype(whh_f.dtype), (1, 0, 2))

    return pl.pallas_call(
        _batched_lstm_head_kernel,
        out_shape=jax.ShapeDtypeStruct((n_seq, output_size), jnp.float32),
        grid=(n_seq // B,),
        in_specs=[
            pl.BlockSpec((seq_len, B, input_size), lambda n: (0, n, 0)),
            _full_spec(wih_f),
            _full_spec(whh_f),
            _full_spec(bias_f),
            _full_spec(wlin_f),
            _full_spec(blin_f),
        ],
        out_specs=pl.BlockSpec((B, output_size), lambda n: (n, 0)),
        compiler_params=pltpu.CompilerParams(
            dimension_semantics=("parallel",)),
    )(xt, wih_f, whh_f, bias_f, wlin_f, blin_f)


# speedup vs baseline: 1.3334x; 1.3334x over previous
import jax
import jax.numpy as jnp
from jax import lax
from jax.experimental import pallas as pl
from jax.experimental.pallas import tpu as pltpu

_MXU_ROWS = 256
_UNROLL = 32


def _batched_lstm_head_kernel(xt_ref, wih_ref, whh_ref, b_ref, wlin_ref,
                              blin_ref, out_ref):
    seq_len, batch, _ = xt_ref.shape
    Hp = whh_ref.shape[0]
    wdtype = whh_ref.dtype
    n_grp = max(1, batch // _MXU_ROWS)
    rows = batch // n_grp

    def step(t, carry):
        xt = xt_ref[t]
        new = []
        for j in range(n_grp):
            h2, c = carry[2 * j], carry[2 * j + 1]
            pre = (jnp.dot(xt[j * rows:(j + 1) * rows], wih_ref[...],
                           preferred_element_type=jnp.float32)
                   + jnp.dot(h2.astype(wdtype), whh_ref[...],
                             preferred_element_type=jnp.float32)
                   + b_ref[...])

            tg = jnp.tanh(pre[:, :3 * Hp])
            t_i = tg[:, 0:Hp]
            t_f = tg[:, Hp:2 * Hp]
            t_o = tg[:, 2 * Hp:3 * Hp]
            g_g = jnp.tanh(pre[:, 3 * Hp:])

            c_new = ((c + g_g) + (t_f * c + t_i * g_g)) * 0.5
            tc = jnp.tanh(c_new)
            h2_new = tc + t_o * tc
            new += [h2_new, c_new]
        return tuple(new)

    init = tuple(jnp.zeros((rows, Hp), jnp.float32) for _ in range(2 * n_grp))
    carry = lax.fori_loop(0, seq_len, step, init, unroll=_UNROLL)

    for j in range(n_grp):
        h2_last = carry[2 * j]
        out_ref[j * rows:(j + 1) * rows, :] = (
            jnp.dot(h2_last.astype(wlin_ref.dtype), wlin_ref[...],
                    preferred_element_type=jnp.float32) + blin_ref[...])


def _full_spec(arr):
    ndim = arr.ndim
    return pl.BlockSpec(arr.shape, lambda n: (0,) * ndim)


def _pick_batch(n_seq):
    for b in (512, 256, 128, 64, 32, 16, 8):
        if n_seq % b == 0:
            return b
    return n_seq


@jax.jit
def kernel(xs, wih_f, whh_f, bias_f, wlin_f, blin_f):
    n_seq, seq_len, input_size = xs.shape
    Hp = whh_f.shape[0]
    output_size = wlin_f.shape[1]
    B = _pick_batch(n_seq)

    col = jnp.concatenate([jnp.full((3 * Hp,), 0.5, jnp.float32),
                           jnp.ones((Hp,), jnp.float32)])
    wih_s = (wih_f.astype(jnp.float32) * col).astype(wih_f.dtype)
    whh_s = (whh_f.astype(jnp.float32) * (col * 0.5)).astype(whh_f.dtype)
    b_s = bias_f * col
    wlin_s = (wlin_f.astype(jnp.float32) * 0.5).astype(wlin_f.dtype)

    xt = jnp.transpose(xs.astype(whh_f.dtype), (1, 0, 2))

    return pl.pallas_call(
        _batched_lstm_head_kernel,
        out_shape=jax.ShapeDtypeStruct((n_seq, output_size), jnp.float32),
        grid=(n_seq // B,),
        in_specs=[
            pl.BlockSpec((seq_len, B, input_size), lambda n: (0, n, 0)),
            _full_spec(wih_s),
            _full_spec(whh_s),
            _full_spec(b_s),
            _full_spec(wlin_s),
            _full_spec(blin_f),
        ],
        out_specs=pl.BlockSpec((B, output_size), lambda n: (n, 0)),
        compiler_params=pltpu.CompilerParams(
            dimension_semantics=("parallel",)),
    )(xt, wih_s, whh_s, b_s, wlin_s, blin_f)


# speedup vs baseline: 1.4053x; 1.0539x over previous
import jax
import jax.numpy as jnp
from jax import lax
from jax.experimental import pallas as pl
from jax.experimental.pallas import tpu as pltpu

_MXU_ROWS = 256
_UNROLL = 32


def _batched_lstm_head_kernel(xt_ref, wih_ref, whh_ref, b_ref, wlin_ref,
                              blin_ref, out_ref):
    seq_len, batch, _ = xt_ref.shape
    Hp = whh_ref.shape[0]
    wdtype = whh_ref.dtype
    n_grp = max(1, batch // _MXU_ROWS)
    rows = batch // n_grp

    col = jax.lax.broadcasted_iota(jnp.int32, (1, 4 * Hp), 1) < 3 * Hp
    colf = jnp.where(col, 0.5, 1.0)
    wih_s = (wih_ref[...].astype(jnp.float32) * colf).astype(wdtype)
    whh_s = (whh_ref[...].astype(jnp.float32) * (colf * 0.5)).astype(wdtype)
    b_s = b_ref[...] * colf

    def step(t, carry):
        xt = xt_ref[t]
        new = []
        for j in range(n_grp):
            h2, c = carry[2 * j], carry[2 * j + 1]
            pre = (jnp.dot(xt[j * rows:(j + 1) * rows], wih_s,
                           preferred_element_type=jnp.float32)
                   + jnp.dot(h2.astype(wdtype), whh_s,
                             preferred_element_type=jnp.float32)
                   + b_s)

            tg = jnp.tanh(pre[:, :3 * Hp])
            t_i = tg[:, 0:Hp]
            t_f = tg[:, Hp:2 * Hp]
            t_o = tg[:, 2 * Hp:3 * Hp]
            g_g = jnp.tanh(pre[:, 3 * Hp:])

            c_new = ((c + g_g) + (t_f * c + t_i * g_g)) * 0.5
            tc = jnp.tanh(c_new)
            h2_new = tc + t_o * tc
            new += [h2_new, c_new]
        return tuple(new)

    init = tuple(jnp.zeros((rows, Hp), jnp.float32) for _ in range(2 * n_grp))
    carry = lax.fori_loop(0, seq_len, step, init, unroll=_UNROLL)

    for j in range(n_grp):
        h2_last = carry[2 * j] * 0.5
        out_ref[j * rows:(j + 1) * rows, :] = (
            jnp.dot(h2_last.astype(wlin_ref.dtype), wlin_ref[...],
                    preferred_element_type=jnp.float32) + blin_ref[...])


def _full_spec(arr):
    ndim = arr.ndim
    return pl.BlockSpec(arr.shape, lambda n: (0,) * ndim)


def _pick_batch(n_seq):
    for b in (512, 256, 128, 64, 32, 16, 8):
        if n_seq % b == 0:
            return b
    return n_seq


@jax.jit
def kernel(xs, wih_f, whh_f, bias_f, wlin_f, blin_f):
    n_seq, seq_len, input_size = xs.shape
    Hp = whh_f.shape[0]
    output_size = wlin_f.shape[1]
    B = _pick_batch(n_seq)

    xt = jnp.transpose(xs.astype(whh_f.dtype), (1, 0, 2))

    return pl.pallas_call(
        _batched_lstm_head_kernel,
        out_shape=jax.ShapeDtypeStruct((n_seq, output_size), jnp.float32),
        grid=(n_seq // B,),
        in_specs=[
            pl.BlockSpec((seq_len, B, input_size), lambda n: (0, n, 0)),
            _full_spec(wih_f),
            _full_spec(whh_f),
            _full_spec(bias_f),
            _full_spec(wlin_f),
            _full_spec(blin_f),
        ],
        out_specs=pl.BlockSpec((B, output_size), lambda n: (n, 0)),
        compiler_params=pltpu.CompilerParams(
            dimension_semantics=("parallel",)),
    )(xt, wih_f, whh_f, bias_f, wlin_f, blin_f)


# speedup vs baseline: 1.4313x; 1.0185x over previous
import jax
import jax.numpy as jnp
from jax import lax
from jax.experimental import pallas as pl
from jax.experimental.pallas import tpu as pltpu

_MXU_ROWS = 256
_UNROLL = 32


def _batched_lstm_head_kernel(xt_ref, wih_ref, whh_ref, b_ref, wlin_ref,
                              blin_ref, out_ref):
    seq_len, batch, _ = xt_ref.shape
    Hp = whh_ref.shape[0]
    wdtype = whh_ref.dtype
    n_grp = max(1, batch // _MXU_ROWS)
    rows = batch // n_grp

    col = jax.lax.broadcasted_iota(jnp.int32, (1, 4 * Hp), 1) < 3 * Hp
    colf = jnp.where(col, 0.5, 1.0)
    wih_s = (wih_ref[...].astype(jnp.float32) * colf).astype(wdtype)
    whh_s = (whh_ref[...].astype(jnp.float32) * colf).astype(wdtype)
    b_s = b_ref[...] * colf

    def step(t, carry):
        xt = xt_ref[t]
        new = []
        for j in range(n_grp):
            h, c = carry[2 * j], carry[2 * j + 1]
            pre = (jnp.dot(xt[j * rows:(j + 1) * rows], wih_s,
                           preferred_element_type=jnp.float32)
                   + jnp.dot(h.astype(wdtype), whh_s,
                             preferred_element_type=jnp.float32)
                   + b_s)

            sig = jnp.tanh(pre[:, :3 * Hp]) * 0.5 + 0.5
            i_g = sig[:, 0:Hp]
            f_g = sig[:, Hp:2 * Hp]
            o_g = sig[:, 2 * Hp:3 * Hp]
            g_g = jnp.tanh(pre[:, 3 * Hp:])

            c_new = f_g * c + i_g * g_g
            h_new = o_g * jnp.tanh(c_new)
            new += [h_new, c_new]
        return tuple(new)

    init = tuple(jnp.zeros((rows, Hp), jnp.float32) for _ in range(2 * n_grp))
    carry = lax.fori_loop(0, seq_len, step, init, unroll=_UNROLL)

    for j in range(n_grp):
        h_last = carry[2 * j]
        out_ref[j * rows:(j + 1) * rows, :] = (
            jnp.dot(h_last.astype(wlin_ref.dtype), wlin_ref[...],
                    preferred_element_type=jnp.float32) + blin_ref[...])


def _full_spec(arr):
    ndim = arr.ndim
    return pl.BlockSpec(arr.shape, lambda n: (0,) * ndim)


def _pick_batch(n_seq):
    for b in (512, 256, 128, 64, 32, 16, 8):
        if n_seq % b == 0:
            return b
    return n_seq


@jax.jit
def kernel(xs, wih_f, whh_f, bias_f, wlin_f, blin_f):
    n_seq, seq_len, input_size = xs.shape
    Hp = whh_f.shape[0]
    output_size = wlin_f.shape[1]
    B = _pick_batch(n_seq)

    xt = jnp.transpose(xs.astype(whh_f.dtype), (1, 0, 2))

    return pl.pallas_call(
        _batched_lstm_head_kernel,
        out_shape=jax.ShapeDtypeStruct((n_seq, output_size), jnp.float32),
        grid=(n_seq // B,),
        in_specs=[
            pl.BlockSpec((seq_len, B, input_size), lambda n: (0, n, 0)),
            _full_spec(wih_f),
            _full_spec(whh_f),
            _full_spec(bias_f),
            _full_spec(wlin_f),
            _full_spec(blin_f),
        ],
        out_specs=pl.BlockSpec((B, output_size), lambda n: (n, 0)),
        compiler_params=pltpu.CompilerParams(
            dimension_semantics=("parallel",)),
    )(xt, wih_f, whh_f, bias_f, wlin_f, blin_f)


# speedup vs baseline: 1.5989x; 1.1171x over previous
import jax
import jax.numpy as jnp
from jax import lax
from jax.experimental import pallas as pl
from jax.experimental.pallas import tpu as pltpu

_MXU_ROWS = 256
_UNROLL = 32


def _batched_lstm_head_kernel(x_ref, wih_ref, whh_ref, b_ref, wlin_ref,
                              blin_ref, out_ref, xt_ref):
    batch, seq_len, _ = x_ref.shape
    Hp = whh_ref.shape[0]
    wdtype = whh_ref.dtype
    n_grp = max(1, batch // _MXU_ROWS)
    rows = batch // n_grp

    xt_ref[...] = jnp.swapaxes(x_ref[...].astype(wdtype), 0, 1)

    col = jax.lax.broadcasted_iota(jnp.int32, (1, 4 * Hp), 1) < 3 * Hp
    colf = jnp.where(col, 0.5, 1.0)
    wih_s = (wih_ref[...].astype(jnp.float32) * colf).astype(wdtype)
    whh_s = (whh_ref[...].astype(jnp.float32) * colf).astype(wdtype)
    b_s = b_ref[...] * colf

    def step(t, carry):
        xt = xt_ref[t]
        new = []
        for j in range(n_grp):
            h, c = carry[2 * j], carry[2 * j + 1]
            pre = (jnp.dot(xt[j * rows:(j + 1) * rows], wih_s,
                           preferred_element_type=jnp.float32)
                   + jnp.dot(h.astype(wdtype), whh_s,
                             preferred_element_type=jnp.float32)
                   + b_s)

            sig = jnp.tanh(pre[:, :3 * Hp]) * 0.5 + 0.5
            i_g = sig[:, 0:Hp]
            f_g = sig[:, Hp:2 * Hp]
            o_g = sig[:, 2 * Hp:3 * Hp]
            g_g = jnp.tanh(pre[:, 3 * Hp:])

            c_new = f_g * c + i_g * g_g
            h_new = o_g * jnp.tanh(c_new)
            new += [h_new, c_new]
        return tuple(new)

    init = tuple(jnp.zeros((rows, Hp), jnp.float32) for _ in range(2 * n_grp))
    carry = lax.fori_loop(0, seq_len, step, init, unroll=_UNROLL)

    for j in range(n_grp):
        h_last = carry[2 * j]
        out_ref[j * rows:(j + 1) * rows, :] = (
            jnp.dot(h_last.astype(wlin_ref.dtype), wlin_ref[...],
                    preferred_element_type=jnp.float32) + blin_ref[...])


def _full_spec(arr):
    ndim = arr.ndim
    return pl.BlockSpec(arr.shape, lambda n: (0,) * ndim)


def _pick_batch(n_seq):
    for b in (512, 256, 128, 64, 32, 16, 8):
        if n_seq % b == 0:
            return b
    return n_seq


@jax.jit
def kernel(xs, wih_f, whh_f, bias_f, wlin_f, blin_f):
    n_seq, seq_len, input_size = xs.shape
    Hp = whh_f.shape[0]
    output_size = wlin_f.shape[1]
    B = _pick_batch(n_seq)

    return pl.pallas_call(
        _batched_lstm_head_kernel,
        out_shape=jax.ShapeDtypeStruct((n_seq, output_size), jnp.float32),
        grid=(n_seq // B,),
        in_specs=[
            pl.BlockSpec((B, seq_len, input_size), lambda n: (n, 0, 0)),
            _full_spec(wih_f),
            _full_spec(whh_f),
            _full_spec(bias_f),
            _full_spec(wlin_f),
            _full_spec(blin_f),
        ],
        out_specs=pl.BlockSpec((B, output_size), lambda n: (n, 0)),
        scratch_shapes=[
            pltpu.VMEM((seq_len, B, input_size), whh_f.dtype),
        ],
        compiler_params=pltpu.CompilerParams(
            dimension_semantics=("parallel",)),
    )(xs, wih_f, whh_f, bias_f, wlin_f, blin_f)


# speedup vs baseline: 1.7309x; 1.0826x over previous
import jax
import jax.numpy as jnp
from jax import lax
from jax.experimental import pallas as pl
from jax.experimental.pallas import tpu as pltpu

_MXU_ROWS = 256
_TCHUNK = 16


def _lstm_chunk_kernel(x_ref, wih_ref, whh_ref, b_ref, wlin_ref,
                       blin_ref, out_ref, xt_ref, h_ref, c_ref):
    batch, u, _ = x_ref.shape
    Hp = whh_ref.shape[0]
    wdtype = whh_ref.dtype
    n_grp = max(1, batch // _MXU_ROWS)
    rows = batch // n_grp
    ci = pl.program_id(1)
    n_c = pl.num_programs(1)

    col = jax.lax.broadcasted_iota(jnp.int32, (1, 4 * Hp), 1) < 3 * Hp
    colf = jnp.where(col, 0.5, 1.0)
    wih_s = (wih_ref[...].astype(jnp.float32) * colf).astype(wdtype)
    whh_s = (whh_ref[...].astype(jnp.float32) * colf).astype(wdtype)
    b_s = b_ref[...] * colf

    @pl.when(ci == 0)
    def _init():
        h_ref[...] = jnp.zeros_like(h_ref)
        c_ref[...] = jnp.zeros_like(c_ref)

    xt_ref[...] = jnp.swapaxes(x_ref[...].astype(wdtype), 0, 1)

    carry = []
    for j in range(n_grp):
        carry += [h_ref[j * rows:(j + 1) * rows, :],
                  c_ref[j * rows:(j + 1) * rows, :]]

    for k in range(u):
        xt = xt_ref[k]
        new = []
        for j in range(n_grp):
            h, c = carry[2 * j], carry[2 * j + 1]
            pre = (jnp.dot(xt[j * rows:(j + 1) * rows], wih_s,
                           preferred_element_type=jnp.float32)
                   + jnp.dot(h.astype(wdtype), whh_s,
                             preferred_element_type=jnp.float32)
                   + b_s)
            sig = jnp.tanh(pre[:, :3 * Hp]) * 0.5 + 0.5
            i_g = sig[:, 0:Hp]
            f_g = sig[:, Hp:2 * Hp]
            o_g = sig[:, 2 * Hp:3 * Hp]
            g_g = jnp.tanh(pre[:, 3 * Hp:])
            c_new = f_g * c + i_g * g_g
            h_new = o_g * jnp.tanh(c_new)
            new += [h_new, c_new]
        carry = new

    for j in range(n_grp):
        h_ref[j * rows:(j + 1) * rows, :] = carry[2 * j]
        c_ref[j * rows:(j + 1) * rows, :] = carry[2 * j + 1]

    @pl.when(ci == n_c - 1)
    def _head():
        for j in range(n_grp):
            out_ref[j * rows:(j + 1) * rows, :] = (
                jnp.dot(carry[2 * j].astype(wlin_ref.dtype), wlin_ref[...],
                        preferred_element_type=jnp.float32) + blin_ref[...])


def _full_spec(arr):
    nd = arr.ndim
    return pl.BlockSpec(arr.shape, lambda n, t: (0,) * nd)


def _pick_batch(n_seq):
    for b in (512, 256, 128, 64, 32, 16, 8):
        if n_seq % b == 0:
            return b
    return n_seq


@jax.jit
def kernel(xs, wih_f, whh_f, bias_f, wlin_f, blin_f):
    n_seq, seq_len, input_size = xs.shape
    Hp = whh_f.shape[0]
    output_size = wlin_f.shape[1]
    B = _pick_batch(n_seq)
    u = _TCHUNK if seq_len % _TCHUNK == 0 else 1

    return pl.pallas_call(
        _lstm_chunk_kernel,
        out_shape=jax.ShapeDtypeStruct((n_seq, output_size), jnp.float32),
        grid=(n_seq // B, seq_len // u),
        in_specs=[
            pl.BlockSpec((B, u, input_size), lambda n, t: (n, t, 0)),
            _full_spec(wih_f),
            _full_spec(whh_f),
            _full_spec(bias_f),
            _full_spec(wlin_f),
            _full_spec(blin_f),
        ],
        out_specs=pl.BlockSpec((B, output_size), lambda n, t: (n, 0)),
        scratch_shapes=[
            pltpu.VMEM((u, B, input_size), whh_f.dtype),
            pltpu.VMEM((B, Hp), jnp.float32),
            pltpu.VMEM((B, Hp), jnp.float32),
        ],
        compiler_params=pltpu.CompilerParams(
            dimension_semantics=("parallel", "arbitrary")),
    )(xs, wih_f, whh_f, bias_f, wlin_f, blin_f)


# speedup vs baseline: 1.7354x; 1.0026x over previous
import jax
import jax.numpy as jnp
from jax import lax
from jax.experimental import pallas as pl
from jax.experimental.pallas import tpu as pltpu

_MXU_ROWS = 256
_TCHUNK = 16


def _lstm_chunk_kernel(x_ref, wih_ref, whh_ref, b_ref, wlin_ref,
                       blin_ref, out_ref, xt_ref, h_ref, c_ref,
                       wih_s_ref, whh_s_ref):
    batch, u, _ = x_ref.shape
    Hp = whh_ref.shape[0]
    wdtype = whh_ref.dtype
    n_grp = max(1, batch // _MXU_ROWS)
    rows = batch // n_grp
    ci = pl.program_id(1)
    n_c = pl.num_programs(1)

    col = jax.lax.broadcasted_iota(jnp.int32, (1, 4 * Hp), 1) < 3 * Hp
    colf = jnp.where(col, 0.5, 1.0)
    b_s = b_ref[...] * colf

    @pl.when(ci == 0)
    def _init():
        wih_s_ref[...] = (wih_ref[...].astype(jnp.float32) * colf).astype(wdtype)
        whh_s_ref[...] = (whh_ref[...].astype(jnp.float32) * colf).astype(wdtype)
        h_ref[...] = jnp.zeros_like(h_ref)
        c_ref[...] = jnp.zeros_like(c_ref)

    wih_s = wih_s_ref[...]
    whh_s = whh_s_ref[...]

    xt_ref[...] = jnp.swapaxes(x_ref[...].astype(wdtype), 0, 1)

    carry = []
    for j in range(n_grp):
        carry += [h_ref[j * rows:(j + 1) * rows, :],
                  c_ref[j * rows:(j + 1) * rows, :]]

    for k in range(u):
        xt = xt_ref[k]
        new = []
        for j in range(n_grp):
            h, c = carry[2 * j], carry[2 * j + 1]
            pre = (jnp.dot(xt[j * rows:(j + 1) * rows], wih_s,
                           preferred_element_type=jnp.float32)
                   + jnp.dot(h.astype(wdtype), whh_s,
                             preferred_element_type=jnp.float32)
                   + b_s)
            sig = jnp.tanh(pre[:, :3 * Hp]) * 0.5 + 0.5
            i_g = sig[:, 0:Hp]
            f_g = sig[:, Hp:2 * Hp]
            o_g = sig[:, 2 * Hp:3 * Hp]
            g_g = jnp.tanh(pre[:, 3 * Hp:])
            c_new = f_g * c + i_g * g_g
            h_new = o_g * jnp.tanh(c_new)
            new += [h_new, c_new]
        carry = new

    for j in range(n_grp):
        h_ref[j * rows:(j + 1) * rows, :] = carry[2 * j]
        c_ref[j * rows:(j + 1) * rows, :] = carry[2 * j + 1]

    @pl.when(ci == n_c - 1)
    def _head():
        for j in range(n_grp):
            out_ref[j * rows:(j + 1) * rows, :] = (
                jnp.dot(carry[2 * j].astype(wlin_ref.dtype), wlin_ref[...],
                        preferred_element_type=jnp.float32) + blin_ref[...])


def _full_spec(arr):
    nd = arr.ndim
    return pl.BlockSpec(arr.shape, lambda n, t: (0,) * nd)


def _pick_batch(n_seq):
    for b in (512, 256, 128, 64, 32, 16, 8):
        if n_seq % b == 0:
            return b
    return n_seq


@jax.jit
def kernel(xs, wih_f, whh_f, bias_f, wlin_f, blin_f):
    n_seq, seq_len, input_size = xs.shape
    Hp = whh_f.shape[0]
    output_size = wlin_f.shape[1]
    B = _pick_batch(n_seq)
    u = _TCHUNK if seq_len % _TCHUNK == 0 else 1

    return pl.pallas_call(
        _lstm_chunk_kernel,
        out_shape=jax.ShapeDtypeStruct((n_seq, output_size), jnp.float32),
        grid=(n_seq // B, seq_len // u),
        in_specs=[
            pl.BlockSpec((B, u, input_size), lambda n, t: (n, t, 0)),
            _full_spec(wih_f),
            _full_spec(whh_f),
            _full_spec(bias_f),
            _full_spec(wlin_f),
            _full_spec(blin_f),
        ],
        out_specs=pl.BlockSpec((B, output_size), lambda n, t: (n, 0)),
        scratch_shapes=[
            pltpu.VMEM((u, B, input_size), whh_f.dtype),
            pltpu.VMEM((B, Hp), jnp.float32),
            pltpu.VMEM((B, Hp), jnp.float32),
            pltpu.VMEM((input_size, 4 * Hp), whh_f.dtype),
            pltpu.VMEM((Hp, 4 * Hp), whh_f.dtype),
        ],
        compiler_params=pltpu.CompilerParams(
            dimension_semantics=("parallel", "arbitrary")),
    )(xs, wih_f, whh_f, bias_f, wlin_f, blin_f)
